# Initial kernel scaffold; baseline (speedup 1.0000x reference)
import jax
import jax.numpy as jnp
from jax.experimental import pallas as pl


def kernel(x, W, a_src, a_dst, bias):
    raise NotImplementedError("write your pallas kernel here")


# speedup vs baseline: 1.9018x; 1.9018x over previous
import jax
import jax.numpy as jnp
from jax.experimental import pallas as pl
from jax.experimental.pallas import tpu as pltpu

B, N, D, O = 32, 512, 128, 128


def _fused_gat_kernel(x_ref, w_ref, a_src_ref, a_dst_ref, bias_ref, out_ref):
    x = x_ref[0]
    W = w_ref[...]
    H = jnp.dot(x, W, preferred_element_type=jnp.float32)

    s = jnp.sum(H * a_src_ref[...], axis=1)
    d = jnp.sum(H * a_dst_ref[...], axis=1)

    E = d[:, None] + s[None, :]
    E = jnp.where(E >= 0, E, 0.2 * E)

    m = jnp.max(E, axis=1, keepdims=True)
    P = jnp.exp(E - m)
    attn = P / jnp.sum(P, axis=1, keepdims=True)

    out = jnp.dot(attn, H, preferred_element_type=jnp.float32) + bias_ref[...]
    out_ref[0] = jnp.where(out > 0, out, jnp.exp(out) - 1.0)


def kernel(x, W, a_src, a_dst, bias):
    a_src2 = a_src.reshape(1, O)
    a_dst2 = a_dst.reshape(1, O)
    bias2 = bias.reshape(1, O)

    return pl.pallas_call(
        _fused_gat_kernel,
        grid=(B,),
        in_specs=[
            pl.BlockSpec((1, N, D), lambda b: (b, 0, 0)),
            pl.BlockSpec((D, O), lambda b: (0, 0)),
            pl.BlockSpec((1, O), lambda b: (0, 0)),
            pl.BlockSpec((1, O), lambda b: (0, 0)),
            pl.BlockSpec((1, O), lambda b: (0, 0)),
        ],
        out_specs=pl.BlockSpec((1, N, O), lambda b: (b, 0, 0)),
        out_shape=jax.ShapeDtypeStruct((B, N, O), jnp.float32),
        compiler_params=pltpu.CompilerParams(
            dimension_semantics=("parallel",),
        ),
    )(x, W, a_src2, a_dst2, bias2)
